# Initial kernel scaffold; baseline (speedup 1.0000x reference)
import jax
import jax.numpy as jnp
from jax.experimental import pallas as pl


def kernel(cls_logits, bbox_regression, anchors):
    raise NotImplementedError("write your pallas kernel here")


# speedup vs baseline: 1.8386x; 1.8386x over previous
import math

import jax
import jax.numpy as jnp
from jax.experimental import pallas as pl

_N = 20000
_NUM_CLASSES = 80
_SCORE_THRESH = 0.05
_NMS_THRESH = 0.5
_DETECTIONS_PER_IMG = 300
_TOPK = 1000
_IMG_H, _IMG_W = 800.0, 1333.0
_BBOX_XFORM_CLIP = math.log(1000.0 / 16.0)
_PAD = 1024


def _score_body(logits_ref, out_ref):
    s = jax.nn.sigmoid(logits_ref[...])
    out_ref[...] = jnp.where(s > _SCORE_THRESH, s, -1e10)


def _nms_body(rel_ref, anc_ref, sc_ref, lab_ref, ob_ref, os_ref, ol_ref):
    rel = rel_ref[...]
    anc = anc_ref[...]
    s0 = sc_ref[...]
    labels = lab_ref[...]

    widths = anc[:, 2:3] - anc[:, 0:1]
    heights = anc[:, 3:4] - anc[:, 1:2]
    ctr_x = anc[:, 0:1] + 0.5 * widths
    ctr_y = anc[:, 1:2] + 0.5 * heights
    dx = rel[:, 0:1]
    dy = rel[:, 1:2]
    dw = jnp.minimum(rel[:, 2:3], _BBOX_XFORM_CLIP)
    dh = jnp.minimum(rel[:, 3:4], _BBOX_XFORM_CLIP)
    pcx = dx * widths + ctr_x
    pcy = dy * heights + ctr_y
    pw = jnp.exp(dw) * widths
    ph = jnp.exp(dh) * heights
    x1 = jnp.clip(pcx - 0.5 * pw, 0.0, _IMG_W)
    y1 = jnp.clip(pcy - 0.5 * ph, 0.0, _IMG_H)
    x2 = jnp.clip(pcx + 0.5 * pw, 0.0, _IMG_W)
    y2 = jnp.clip(pcy + 0.5 * ph, 0.0, _IMG_H)
    boxes = jnp.concatenate([x1, y1, x2, y2], axis=1)

    max_coord = jnp.max(boxes)
    off = labels.astype(jnp.float32) * (max_coord + 1.0)
    bo = boxes + off
    areas = (bo[:, 2:3] - bo[:, 0:1]) * (bo[:, 3:4] - bo[:, 1:2])
    iota = jax.lax.broadcasted_iota(jnp.int32, (_PAD, 1), 0)

    def body(i, s):
        m = jnp.max(s)
        idx = jnp.min(jnp.where(s == m, iota, _PAD))
        valid = m > -1e9
        sel = iota == idx
        selm = sel.astype(jnp.float32)
        bx1 = jnp.sum(bo[:, 0:1] * selm)
        by1 = jnp.sum(bo[:, 1:2] * selm)
        bx2 = jnp.sum(bo[:, 2:3] * selm)
        by2 = jnp.sum(bo[:, 3:4] * selm)
        a_sel = jnp.sum(areas * selm)
        box_row = jnp.sum(boxes * selm, axis=0, keepdims=True)
        lab_sel = jnp.sum(jnp.where(sel, labels, 0))

        xx1 = jnp.maximum(bx1, bo[:, 0:1])
        yy1 = jnp.maximum(by1, bo[:, 1:2])
        xx2 = jnp.minimum(bx2, bo[:, 2:3])
        yy2 = jnp.minimum(by2, bo[:, 3:4])
        inter = jnp.maximum(xx2 - xx1, 0.0) * jnp.maximum(yy2 - yy1, 0.0)
        iou = inter / (a_sel + areas - inter + 1e-9)
        s = jnp.where(iou > _NMS_THRESH, -1e10, s)
        s = jnp.where(sel, -1e10, s)

        ob_ref[pl.ds(i, 1), :] = jnp.where(valid, box_row, 0.0)
        os_ref[pl.ds(i, 1), :] = jnp.full((1, 1), 0.0) + jnp.where(valid, m, 0.0)
        ol_ref[pl.ds(i, 1), :] = jnp.full((1, 1), 0, jnp.int32) + jnp.where(
            valid, lab_sel, -1
        )
        return s

    jax.lax.fori_loop(0, _DETECTIONS_PER_IMG, body, s0)


def kernel(cls_logits, bbox_regression, anchors):
    masked = pl.pallas_call(
        _score_body,
        out_shape=jax.ShapeDtypeStruct((_N, _NUM_CLASSES), jnp.float32),
    )(cls_logits)

    top_scores, topk_idxs = jax.lax.top_k(masked.reshape(-1), _TOPK)
    anchor_idxs = topk_idxs // _NUM_CLASSES
    labels = topk_idxs % _NUM_CLASSES

    rel = jnp.zeros((_PAD, 4), jnp.float32).at[:_TOPK].set(
        bbox_regression[anchor_idxs]
    )
    anc = jnp.zeros((_PAD, 4), jnp.float32).at[:_TOPK].set(anchors[anchor_idxs])
    sc = jnp.full((_PAD, 1), -1e10, jnp.float32).at[:_TOPK, 0].set(top_scores)
    lab = jnp.zeros((_PAD, 1), jnp.int32).at[:_TOPK, 0].set(labels)

    out_boxes, out_scores, out_labels = pl.pallas_call(
        _nms_body,
        out_shape=[
            jax.ShapeDtypeStruct((_DETECTIONS_PER_IMG, 4), jnp.float32),
            jax.ShapeDtypeStruct((_DETECTIONS_PER_IMG, 1), jnp.float32),
            jax.ShapeDtypeStruct((_DETECTIONS_PER_IMG, 1), jnp.int32),
        ],
    )(rel, anc, sc, lab)

    return out_boxes, out_scores.reshape(-1), out_labels.reshape(-1)
